# baseline (speedup 1.0000x reference)
import functools

import jax
import jax.numpy as jnp
from jax import lax
from jax.experimental import pallas as pl
from jax.experimental.pallas import tpu as pltpu
from jax.experimental.pallas import tpu_sc as plsc

NC = 2
NS = 16
L = 16


@functools.lru_cache(maxsize=None)
def _build_sc_gather(vocab: int, num_heads: int, area: int):
  nw = NC * NS
  area2 = area * area
  chunk = area2 // nw
  nrows = chunk // area
  piece = 512
  npiece = chunk // piece
  pvec = piece // L
  per_row = area // piece

  mesh = plsc.VectorSubcoreMesh(core_axis_name="c", subcore_axis_name="s")

  @functools.partial(
      pl.kernel,
      out_type=jax.ShapeDtypeStruct((1, num_heads, area, area), jnp.float32),
      mesh=mesh,
      scratch_types=[
          pltpu.VMEM((nrows, area), jnp.int32),
          pltpu.VMEM((vocab * num_heads,), jnp.float32),
          pltpu.VMEM((num_heads, piece), jnp.float32),
          pltpu.VMEM((num_heads, piece), jnp.float32),
          pltpu.SemaphoreType.DMA,
          pltpu.SemaphoreType.DMA,
      ],
      compiler_params=pltpu.CompilerParams(needs_layout_passes=False),
  )
  def sc_gather(table_hbm, idx_hbm, out_hbm, idx_v, tab_v, out_v0, out_v1,
                sem0, sem1):
    wid = lax.axis_index("s") * NC + lax.axis_index("c")
    row0 = wid * nrows
    pltpu.sync_copy(idx_hbm.at[pl.ds(row0, nrows)], idx_v)
    pltpu.sync_copy(table_hbm, tab_v)
    bufs = (out_v0, out_v1)
    sems = (sem0, sem1)

    def out_dst(pp):
      r = row0 + pp // per_row
      c = (pp % per_row) * piece
      return out_hbm.at[0, :, r, pl.ds(c, piece)]

    @pl.loop(0, npiece, step=2)
    def piece_loop(p):
      for k in range(2):
        buf, sem = bufs[k], sems[k]
        pp = p + k
        r = pp // per_row
        c = (pp % per_row) * piece

        @pl.when(pp >= 2)
        def _wait_prev():
          pltpu.make_async_copy(buf, out_dst(pp - 2), sem).wait()

        @plsc.parallel_loop(0, pvec, 1, unroll=4)
        def body(v):
          iv = idx_v[r, pl.ds(c + v * L, L)] * num_heads
          for h in range(num_heads):
            buf[h, pl.ds(v * L, L)] = plsc.load_gather(tab_v, [iv + h])

        pltpu.async_copy(buf, out_dst(pp), sem)

    for k in range(2):
      pltpu.make_async_copy(bufs[k], out_dst(npiece - 2 + k), sems[k]).wait()

  return sc_gather


def kernel(relative_position_bias_table, relative_position_index):
  vocab, num_heads = relative_position_bias_table.shape
  area = relative_position_index.shape[0]
  table = relative_position_bias_table.astype(jnp.float32).reshape(-1)
  idx = relative_position_index.astype(jnp.int32)
  return _build_sc_gather(vocab, num_heads, area)(table, idx)


# speedup vs baseline: 2.1430x; 2.1430x over previous
import functools

import jax
import jax.numpy as jnp
from jax import lax
from jax.experimental import pallas as pl
from jax.experimental.pallas import tpu as pltpu
from jax.experimental.pallas import tpu_sc as plsc

NC = 2
NS = 16
L = 16


@functools.lru_cache(maxsize=None)
def _build_sc_gather(v_pad: int, num_heads: int, area: int):
  nw = NC * NS
  area2 = area * area
  chunk = area2 // nw
  nrows = chunk // area
  piece = 512
  npiece = chunk // piece
  pvec = piece // L
  per_row = area // piece

  mesh = plsc.VectorSubcoreMesh(core_axis_name="c", subcore_axis_name="s")

  @functools.partial(
      pl.kernel,
      out_type=jax.ShapeDtypeStruct((1, num_heads, area, area), jnp.float32),
      mesh=mesh,
      scratch_types=[
          pltpu.VMEM((nrows, area), jnp.int32),
          pltpu.VMEM((num_heads, v_pad), jnp.float32),
          pltpu.VMEM((num_heads, piece), jnp.float32),
          pltpu.VMEM((num_heads, piece), jnp.float32),
          pltpu.SemaphoreType.DMA,
          pltpu.SemaphoreType.DMA,
      ],
      compiler_params=pltpu.CompilerParams(needs_layout_passes=False),
  )
  def sc_gather(table_t_hbm, idx_hbm, out_hbm, idx_v, tab_v, out_v0, out_v1,
                sem0, sem1):
    wid = lax.axis_index("s") * NC + lax.axis_index("c")
    row0 = wid * nrows
    pltpu.sync_copy(idx_hbm.at[pl.ds(row0, nrows)], idx_v)
    pltpu.sync_copy(table_t_hbm, tab_v)
    bufs = (out_v0, out_v1)
    sems = (sem0, sem1)

    def out_dst(pp):
      r = row0 + pp // per_row
      c = (pp % per_row) * piece
      return out_hbm.at[0, :, r, pl.ds(c, piece)]

    @pl.loop(0, npiece, step=2)
    def piece_loop(p):
      for k in range(2):
        buf, sem = bufs[k], sems[k]
        pp = p + k
        r = pp // per_row
        c = (pp % per_row) * piece

        @pl.when(pp >= 2)
        def _wait_prev():
          pltpu.make_async_copy(buf, out_dst(pp - 2), sem).wait()

        @plsc.parallel_loop(0, pvec, 1, unroll=4)
        def body(v):
          iv = idx_v[r, pl.ds(c + v * L, L)]
          for h in range(num_heads):
            hv = jnp.full((L,), h, jnp.int32)
            buf[h, pl.ds(v * L, L)] = plsc.load_gather(tab_v, [hv, iv])

        pltpu.async_copy(buf, out_dst(pp), sem)

    for k in range(2):
      pltpu.make_async_copy(bufs[k], out_dst(npiece - 2 + k), sems[k]).wait()

  return sc_gather


def kernel(relative_position_bias_table, relative_position_index):
  vocab, num_heads = relative_position_bias_table.shape
  area = relative_position_index.shape[0]
  v_pad = -(-vocab // 8) * 8
  table_t = jnp.zeros((num_heads, v_pad), jnp.float32)
  table_t = table_t.at[:, :vocab].set(relative_position_bias_table.T)
  idx = relative_position_index.astype(jnp.int32)
  return _build_sc_gather(v_pad, num_heads, area)(table_t, idx)


# speedup vs baseline: 2.1634x; 1.0095x over previous
import functools

import jax
import jax.numpy as jnp
from jax import lax
from jax.experimental import pallas as pl
from jax.experimental.pallas import tpu as pltpu
from jax.experimental.pallas import tpu_sc as plsc

NC = 2
NS = 16
L = 16


@functools.lru_cache(maxsize=None)
def _build_sc_gather(v_pad: int, num_heads: int, area: int):
  nw = NC * NS
  area2 = area * area
  chunk = area2 // nw
  nrows = chunk // area
  piece = 512
  npiece = chunk // piece
  pvec = piece // L
  per_row = area // piece

  mesh = plsc.VectorSubcoreMesh(core_axis_name="c", subcore_axis_name="s")

  @functools.partial(
      pl.kernel,
      out_type=jax.ShapeDtypeStruct((1, num_heads, area, area), jnp.float32),
      mesh=mesh,
      scratch_types=[
          pltpu.VMEM((nrows, area), jnp.int32),
          pltpu.VMEM((num_heads, v_pad), jnp.float32),
          pltpu.VMEM((num_heads, piece), jnp.float32),
          pltpu.VMEM((num_heads, piece), jnp.float32),
          pltpu.SemaphoreType.DMA,
          pltpu.SemaphoreType.DMA,
      ],
      compiler_params=pltpu.CompilerParams(needs_layout_passes=False),
  )
  def sc_gather(table_t_hbm, idx_hbm, out_hbm, idx_v, tab_v, out_v0, out_v1,
                sem0, sem1):
    wid = lax.axis_index("s") * NC + lax.axis_index("c")
    row0 = wid * nrows
    idx_cp = pltpu.async_copy(idx_hbm.at[pl.ds(row0, nrows)], idx_v, sem0)
    tab_cp = pltpu.async_copy(table_t_hbm, tab_v, sem1)
    idx_cp.wait()
    tab_cp.wait()
    bufs = (out_v0, out_v1)
    sems = (sem0, sem1)

    def out_dst(pp):
      r = row0 + pp // per_row
      c = (pp % per_row) * piece
      return out_hbm.at[0, :, r, pl.ds(c, piece)]

    @pl.loop(0, npiece, step=2)
    def piece_loop(p):
      for k in range(2):
        buf, sem = bufs[k], sems[k]
        pp = p + k
        r = pp // per_row
        c = (pp % per_row) * piece

        @pl.when(pp >= 2)
        def _wait_prev():
          pltpu.make_async_copy(buf, out_dst(pp - 2), sem).wait()

        @plsc.parallel_loop(0, pvec, 1, unroll=4)
        def body(v):
          iv = idx_v[r, pl.ds(c + v * L, L)]
          for h in range(num_heads):
            hv = jnp.full((L,), h, jnp.int32)
            buf[h, pl.ds(v * L, L)] = plsc.load_gather(tab_v, [hv, iv])

        pltpu.async_copy(buf, out_dst(pp), sem)

    for k in range(2):
      pltpu.make_async_copy(bufs[k], out_dst(npiece - 2 + k), sems[k]).wait()

  return sc_gather


def kernel(relative_position_bias_table, relative_position_index):
  vocab, num_heads = relative_position_bias_table.shape
  area = relative_position_index.shape[0]
  v_pad = -(-vocab // 8) * 8
  table_t = jnp.zeros((num_heads, v_pad), jnp.float32)
  table_t = table_t.at[:, :vocab].set(relative_position_bias_table.T)
  idx = relative_position_index.astype(jnp.int32)
  return _build_sc_gather(v_pad, num_heads, area)(table_t, idx)
